# Initial kernel scaffold; baseline (speedup 1.0000x reference)
import jax
import jax.numpy as jnp
from jax.experimental import pallas as pl


def kernel(ids, table):
    raise NotImplementedError("write your pallas kernel here")


# speedup vs baseline: 1.0935x; 1.0935x over previous
import functools

import jax
import jax.numpy as jnp
from jax import lax
from jax.experimental import pallas as pl
from jax.experimental.pallas import tpu as pltpu
from jax.experimental.pallas import tpu_sc as plsc

D = 32
L = 128
SUB = 8
C = SUB * L


@functools.lru_cache(maxsize=None)
def _make_gather(B, V):
    info = plsc.get_sparse_core_info()
    NC, NS = info.num_cores, info.num_subcores
    NW = NC * NS
    b_per_w = B // NW
    nchunk = b_per_w // C
    assert b_per_w % C == 0

    mesh = plsc.VectorSubcoreMesh(core_axis_name="c", subcore_axis_name="s")

    @functools.partial(
        pl.kernel,
        mesh=mesh,
        out_type=jax.ShapeDtypeStruct((B, D), jnp.float32),
        scratch_types=[
            pltpu.VMEM((SUB, L), jnp.int32),
            pltpu.VMEM((C, D), jnp.float32),
            pltpu.SemaphoreType.DMA,
        ],
        compiler_params=pltpu.CompilerParams(use_tc_tiling_on_sc=False),
    )
    def gather_kernel(ids_hbm, table_hbm, out_hbm, idx_v, rows_v, sem):
        wid = lax.axis_index("s") * NC + lax.axis_index("c")
        base = wid * b_per_w

        def body(g, carry):
            row0 = base + g * C
            ids_row0 = pl.multiple_of(base // L + g * SUB, 8)
            pltpu.sync_copy(ids_hbm.at[pl.ds(ids_row0, SUB), :], idx_v)
            copies = [
                pltpu.async_copy(table_hbm.at[idx_v.at[j]],
                                 rows_v.at[pl.ds(j * L, L)], sem)
                for j in range(SUB)
            ]
            for cp in copies:
                cp.wait()
            pltpu.sync_copy(rows_v, out_hbm.at[pl.ds(row0, C), :])
            return carry

        lax.fori_loop(0, nchunk, body, 0)

    return gather_kernel


def kernel(ids, table):
    bsz, hist = ids.shape
    B = bsz * hist
    ids_flat = ids.reshape(B // L, L).astype(jnp.int32)
    out = _make_gather(B, table.shape[0])(ids_flat, table)
    return out.reshape(bsz, hist, D)


# speedup vs baseline: 1.1125x; 1.0174x over previous
import functools

import jax
import jax.numpy as jnp
from jax import lax
from jax.experimental import pallas as pl
from jax.experimental.pallas import tpu as pltpu
from jax.experimental.pallas import tpu_sc as plsc

D = 32
L = 128
SUB = 10
C = SUB * L
NBUF = 2


@functools.lru_cache(maxsize=None)
def _make_gather(B, V):
    info = plsc.get_sparse_core_info()
    NC, NS = info.num_cores, info.num_subcores
    NW = NC * NS
    b_per_w = B // NW
    nchunk = b_per_w // C
    groups = b_per_w // L
    assert b_per_w % C == 0 and nchunk % NBUF == 0

    mesh = plsc.VectorSubcoreMesh(core_axis_name="c", subcore_axis_name="s")

    @functools.partial(
        pl.kernel,
        mesh=mesh,
        out_type=jax.ShapeDtypeStruct((B, D), jnp.float32),
        scratch_types=[
            pltpu.VMEM((groups, L), jnp.int32),
            *[pltpu.VMEM((C, D), jnp.float32) for _ in range(NBUF)],
            *[pltpu.SemaphoreType.DMA for _ in range(2 * NBUF)],
        ],
        compiler_params=pltpu.CompilerParams(use_tc_tiling_on_sc=False),
    )
    def gather_kernel(ids_hbm, table_hbm, out_hbm, idx_v, *bufs_and_sems):
        rows = bufs_and_sems[:NBUF]
        sem_g = bufs_and_sems[NBUF:2 * NBUF]
        sem_w = bufs_and_sems[2 * NBUF:]
        wid = lax.axis_index("s") * NC + lax.axis_index("c")
        base = wid * b_per_w

        pltpu.sync_copy(
            ids_hbm.at[pl.ds(pl.multiple_of(base // L, 8), groups), :], idx_v)

        def start_gathers(g, b):
            for j in range(SUB):
                pltpu.async_copy(table_hbm.at[idx_v.at[g * SUB + j]],
                                 rows[b].at[pl.ds(j * L, L)], sem_g[b])

        def wait_gathers(b):
            pltpu.make_async_copy(table_hbm.at[pl.ds(0, C)], rows[b],
                                  sem_g[b]).wait()

        def start_writeback(g, b):
            pltpu.make_async_copy(
                rows[b], out_hbm.at[pl.ds(base + g * C, C), :],
                sem_w[b]).start()

        def wait_writeback(b):
            pltpu.make_async_copy(
                rows[b], out_hbm.at[pl.ds(base, C), :], sem_w[b]).wait()

        start_gathers(0, 0)

        def body(s, carry):
            for b in range(NBUF):
                g = s * NBUF + b
                bn = (b + 1) % NBUF
                @pl.when((g + 1 - NBUF >= 0) & (g + 1 < nchunk))
                def _():
                    wait_writeback(bn)

                @pl.when(g + 1 < nchunk)
                def _():
                    start_gathers(g + 1, bn)

                wait_gathers(b)
                start_writeback(g, b)
            return carry

        lax.fori_loop(0, nchunk // NBUF, body, 0)

        for b in range(NBUF):
            wait_writeback(b)

    return gather_kernel


def kernel(ids, table):
    bsz, hist = ids.shape
    B = bsz * hist
    ids_flat = ids.reshape(B // L, L).astype(jnp.int32)
    out = _make_gather(B, table.shape[0])(ids_flat, table)
    return out.reshape(bsz, hist, D)


# speedup vs baseline: 1.8046x; 1.6221x over previous
import functools

import jax
import jax.numpy as jnp
from jax import lax
from jax.experimental import pallas as pl
from jax.experimental.pallas import tpu as pltpu
from jax.experimental.pallas import tpu_sc as plsc

D = 32
H = 50
RC = 16
NBUF = 2


@functools.lru_cache(maxsize=None)
def _make_gather(BS, V):
    info = plsc.get_sparse_core_info()
    NC, NS = info.num_cores, info.num_subcores
    NW = NC * NS
    rows_per_w = BS // NW
    nchunk = rows_per_w // RC
    assert rows_per_w % RC == 0 and nchunk % NBUF == 0

    mesh = plsc.VectorSubcoreMesh(core_axis_name="c", subcore_axis_name="s")

    @functools.partial(
        pl.kernel,
        mesh=mesh,
        out_type=jax.ShapeDtypeStruct((BS, H, D), jnp.float32),
        scratch_types=[
            pltpu.VMEM((rows_per_w, H), jnp.int32),
            *[pltpu.VMEM((RC, H, D), jnp.float32) for _ in range(NBUF)],
            *[pltpu.SemaphoreType.DMA for _ in range(2 * NBUF)],
        ],
        compiler_params=pltpu.CompilerParams(use_tc_tiling_on_sc=False),
    )
    def gather_kernel(ids_hbm, table_hbm, out_hbm, idx_v, *bufs_and_sems):
        rows = bufs_and_sems[:NBUF]
        sem_g = bufs_and_sems[NBUF:2 * NBUF]
        sem_w = bufs_and_sems[2 * NBUF:]
        wid = lax.axis_index("s") * NC + lax.axis_index("c")
        base = wid * rows_per_w

        pltpu.sync_copy(
            ids_hbm.at[pl.ds(pl.multiple_of(base, 8), rows_per_w), :], idx_v)

        def start_gathers(g, b):
            for r in range(RC):
                pltpu.async_copy(table_hbm.at[idx_v.at[g * RC + r]],
                                 rows[b].at[r], sem_g[b])

        def wait_gathers(b):
            pltpu.make_async_copy(
                out_hbm.at[pl.ds(0, RC)], rows[b], sem_g[b]).wait()

        def start_writeback(g, b):
            pltpu.make_async_copy(
                rows[b], out_hbm.at[pl.ds(base + g * RC, RC)],
                sem_w[b]).start()

        def wait_writeback(b):
            pltpu.make_async_copy(
                rows[b], out_hbm.at[pl.ds(base, RC)], sem_w[b]).wait()

        start_gathers(0, 0)

        def body(s, carry):
            for b in range(NBUF):
                g = s * NBUF + b
                bn = (b + 1) % NBUF
                @pl.when((g + 1 - NBUF >= 0) & (g + 1 < nchunk))
                def _():
                    wait_writeback(bn)

                @pl.when(g + 1 < nchunk)
                def _():
                    start_gathers(g + 1, bn)

                wait_gathers(b)
                start_writeback(g, b)
            return carry

        lax.fori_loop(0, nchunk // NBUF, body, 0)

        for b in range(NBUF):
            wait_writeback(b)

    return gather_kernel


def kernel(ids, table):
    bsz, hist = ids.shape
    return _make_gather(bsz, table.shape[0])(ids, table)
